# Initial kernel scaffold; baseline (speedup 1.0000x reference)
import jax
import jax.numpy as jnp
from jax.experimental import pallas as pl


def kernel(s):
    raise NotImplementedError("write your pallas kernel here")


# speedup vs baseline: 24.8012x; 24.8012x over previous
import functools

import jax
import jax.numpy as jnp
from jax import lax
from jax.experimental import pallas as pl
from jax.experimental.pallas import tpu as pltpu
from jax.experimental.pallas import tpu_sc as plsc

B, N, M = 32, 512, 512
L = 16
NC, NS = 2, 16
ROWS_BLK = 64
TC_ROWS = 256


def _p1_body(s_hbm, removed_hbm, rowbuf, pen, rem):
    b = lax.axis_index("s") * NC + lax.axis_index("c")
    lane_iota = lax.broadcasted_iota(jnp.int32, (L,), 0)

    for k in range(M // L):
        pen[pl.ds(k * L, L)] = jnp.zeros((L,), jnp.float32)
        rem[pl.ds(k * L, L)] = jnp.zeros((L,), jnp.int32)

    for blk in range(N // ROWS_BLK):
        pltpu.sync_copy(s_hbm.at[b, pl.ds(blk * ROWS_BLK, ROWS_BLK)], rowbuf)

        def row_body(r, carry, blk=blk):
            i = blk * ROWS_BLK + r
            best_v = jnp.full((L,), -jnp.inf, jnp.float32)
            best_i = jnp.zeros((L,), jnp.int32)
            for k in range(M // L):
                v = rowbuf[r, pl.ds(k * L, L)] + pen[pl.ds(k * L, L)]
                gt = v > best_v
                best_v = jnp.where(gt, v, best_v)
                best_i = jnp.where(gt, lane_iota + (k * L), best_i)
            mx = jnp.max(best_v)
            cand = jnp.where(best_v == mx, best_i, jnp.int32(2**30))
            idx = jnp.min(cand)
            idxv = jnp.full((L,), idx, jnp.int32)
            lane0 = lane_iota == 0
            plsc.store_scatter(
                pen, [idxv], jnp.full((L,), -jnp.inf, jnp.float32), mask=lane0)
            plsc.store_scatter(
                rem, [idxv], jnp.full((L,), i, jnp.int32), mask=lane0)
            return carry

        lax.fori_loop(0, ROWS_BLK, row_body, 0)

    pltpu.sync_copy(rem, removed_hbm.at[b])


def _phase1(s):
    mesh = plsc.VectorSubcoreMesh(core_axis_name="c", subcore_axis_name="s")
    kern = functools.partial(
        pl.kernel,
        mesh=mesh,
        out_type=jax.ShapeDtypeStruct((B, M), jnp.int32),
        scratch_types=[
            pltpu.VMEM((ROWS_BLK, M), jnp.float32),
            pltpu.VMEM((M,), jnp.float32),
            pltpu.VMEM((M,), jnp.int32),
        ],
        compiler_params=pltpu.CompilerParams(needs_layout_passes=False),
    )(_p1_body)
    return kern(s)


def _p2_kernel(s_ref, rem_ref, o_ref):
    rows = s_ref[0]
    ra = rem_ref[0]
    i0 = pl.program_id(1) * TC_ROWS
    row_ids = i0 + lax.broadcasted_iota(jnp.int32, (TC_ROWS, 1), 0)
    mask = ra >= row_ids
    neg = jnp.where(mask, rows, -jnp.inf)
    mx = jnp.max(neg, axis=1, keepdims=True)
    e = jnp.where(mask, jnp.exp(rows - mx), 0.0)
    o_ref[0] = e / jnp.sum(e, axis=1, keepdims=True)


def _phase2(s, removed):
    rem3 = removed.reshape(B, 1, M)
    return pl.pallas_call(
        _p2_kernel,
        grid=(B, N // TC_ROWS),
        in_specs=[
            pl.BlockSpec((1, TC_ROWS, M), lambda b, r: (b, r, 0)),
            pl.BlockSpec((1, 1, M), lambda b, r: (b, 0, 0)),
        ],
        out_specs=pl.BlockSpec((1, TC_ROWS, M), lambda b, r: (b, r, 0)),
        out_shape=jax.ShapeDtypeStruct((B, N, M), jnp.float32),
    )(s, rem3)


def kernel(s):
    removed = _phase1(s)
    return _phase2(s, removed)


# speedup vs baseline: 27.4899x; 1.1084x over previous
import functools

import jax
import jax.numpy as jnp
from jax import lax
from jax.experimental import pallas as pl
from jax.experimental.pallas import tpu as pltpu
from jax.experimental.pallas import tpu_sc as plsc

B, N, M = 32, 512, 512
L = 16
NC, NS = 2, 16
ROWS_BLK = 64
TC_ROWS = 256


def _p1_body(s_hbm, removed_hbm, buf0, buf1, pen, rem, sem0, sem1):
    b = lax.axis_index("s") * NC + lax.axis_index("c")
    lane_iota = lax.broadcasted_iota(jnp.int32, (L,), 0)

    for k in range(M // L):
        pen[pl.ds(k * L, L)] = jnp.zeros((L,), jnp.float32)
        rem[pl.ds(k * L, L)] = jnp.zeros((L,), jnp.int32)

    bufs = (buf0, buf1)
    sems = (sem0, sem1)
    n_blk = N // ROWS_BLK
    copies = [None] * n_blk
    copies[0] = pltpu.async_copy(
        s_hbm.at[b, pl.ds(0, ROWS_BLK)], bufs[0], sems[0])

    n_grp = 4
    per_grp = M // L // n_grp

    for blk in range(n_blk):
        rowbuf = bufs[blk % 2]
        copies[blk].wait()
        if blk + 1 < n_blk:
            copies[blk + 1] = pltpu.async_copy(
                s_hbm.at[b, pl.ds((blk + 1) * ROWS_BLK, ROWS_BLK)],
                bufs[(blk + 1) % 2], sems[(blk + 1) % 2])

        def row_body(r, carry, rowbuf=rowbuf, blk=blk):
            i = blk * ROWS_BLK + r
            accs = []
            for g in range(n_grp):
                bv = jnp.full((L,), -jnp.inf, jnp.float32)
                bi = jnp.zeros((L,), jnp.int32)
                for k in range(per_grp):
                    kk = g * per_grp + k
                    v = rowbuf[r, pl.ds(kk * L, L)] + pen[pl.ds(kk * L, L)]
                    gt = v > bv
                    bv = jnp.where(gt, v, bv)
                    bi = jnp.where(gt, lane_iota + (kk * L), bi)
                accs.append((bv, bi))
            while len(accs) > 1:
                nxt = []
                for (av, ai), (bv, bi) in zip(accs[::2], accs[1::2]):
                    gt = bv > av
                    nxt.append((jnp.where(gt, bv, av), jnp.where(gt, bi, ai)))
                accs = nxt
            best_v, best_i = accs[0]
            mx = jnp.max(best_v)
            cand = jnp.where(best_v == mx, best_i, jnp.int32(2**30))
            idx = jnp.min(cand)
            idxv = jnp.full((L,), idx, jnp.int32)
            lane0 = lane_iota == 0
            plsc.store_scatter(
                pen, [idxv], jnp.full((L,), -jnp.inf, jnp.float32), mask=lane0)
            plsc.store_scatter(
                rem, [idxv], jnp.full((L,), i, jnp.int32), mask=lane0)
            return carry

        lax.fori_loop(0, ROWS_BLK, row_body, 0)

    pltpu.sync_copy(rem, removed_hbm.at[b])


def _phase1(s):
    mesh = plsc.VectorSubcoreMesh(core_axis_name="c", subcore_axis_name="s")
    kern = functools.partial(
        pl.kernel,
        mesh=mesh,
        out_type=jax.ShapeDtypeStruct((B, M), jnp.int32),
        scratch_types=[
            pltpu.VMEM((ROWS_BLK, M), jnp.float32),
            pltpu.VMEM((ROWS_BLK, M), jnp.float32),
            pltpu.VMEM((M,), jnp.float32),
            pltpu.VMEM((M,), jnp.int32),
            pltpu.SemaphoreType.DMA,
            pltpu.SemaphoreType.DMA,
        ],
        compiler_params=pltpu.CompilerParams(needs_layout_passes=False),
    )(_p1_body)
    return kern(s)


def _p2_kernel(s_ref, rem_ref, o_ref):
    rows = s_ref[0]
    ra = rem_ref[0]
    i0 = pl.program_id(1) * TC_ROWS
    row_ids = i0 + lax.broadcasted_iota(jnp.int32, (TC_ROWS, 1), 0)
    mask = ra >= row_ids
    neg = jnp.where(mask, rows, -jnp.inf)
    mx = jnp.max(neg, axis=1, keepdims=True)
    e = jnp.where(mask, jnp.exp(rows - mx), 0.0)
    o_ref[0] = e / jnp.sum(e, axis=1, keepdims=True)


def _phase2(s, removed):
    rem3 = removed.reshape(B, 1, M)
    return pl.pallas_call(
        _p2_kernel,
        grid=(B, N // TC_ROWS),
        in_specs=[
            pl.BlockSpec((1, TC_ROWS, M), lambda b, r: (b, r, 0)),
            pl.BlockSpec((1, 1, M), lambda b, r: (b, 0, 0)),
        ],
        out_specs=pl.BlockSpec((1, TC_ROWS, M), lambda b, r: (b, r, 0)),
        out_shape=jax.ShapeDtypeStruct((B, N, M), jnp.float32),
    )(s, rem3)


def kernel(s):
    removed = _phase1(s)
    return _phase2(s, removed)
